# baseline (speedup 1.0000x reference)
import functools

import jax
import jax.numpy as jnp
from jax import lax
from jax.experimental import pallas as pl
from jax.experimental.pallas import tpu as pltpu
from jax.experimental.pallas import tpu_sc as plsc

N_POS_EMB = 512
N_HEADS = 16
TROWS = N_POS_EMB + 1

NC = 2
NS = 16
NW = NC * NS
LANES = 16


@jax.jit
def _embed(d_mat, table_t_flat):
    _, R, C = d_mat.shape
    rows_per_w = R // NW
    mesh = plsc.VectorSubcoreMesh(core_axis_name="c", subcore_axis_name="s")

    @functools.partial(
        pl.kernel,
        out_type=jax.ShapeDtypeStruct((1, R, N_HEADS, C), jnp.float32),
        mesh=mesh,
        scratch_types=[
            pltpu.VMEM((N_HEADS * TROWS,), jnp.float32),
            pltpu.VMEM((2, C), jnp.float32),
            pltpu.VMEM((2, N_HEADS, C), jnp.float32),
            pltpu.SemaphoreType.DMA,
            pltpu.SemaphoreType.DMA,
            pltpu.SemaphoreType.DMA,
            pltpu.SemaphoreType.DMA,
        ],
        compiler_params=pltpu.CompilerParams(needs_layout_passes=False),
    )
    def k(x_hbm, tab_hbm, out_hbm, tab_v, x_v, ob_v, si0, si1, so0, so1):
        wid = lax.axis_index("s") * NC + lax.axis_index("c")
        row0 = wid * rows_per_w
        last = row0 + rows_per_w - 1

        pltpu.sync_copy(tab_hbm, tab_v)
        pltpu.async_copy(x_hbm.at[0, row0], x_v.at[0], si0)
        pltpu.async_copy(x_hbm.at[0, row0 + 1], x_v.at[1], si1)

        def compute(p):
            @plsc.parallel_loop(0, C // LANES, 1, unroll=8)
            def _(j):
                x = x_v[p, pl.ds(j * LANES, LANES)]
                v = jnp.minimum(x, 1.0) * float(N_POS_EMB)
                idx = v.astype(jnp.int32)
                for h in range(N_HEADS):
                    g = plsc.load_gather(tab_v, [idx + (h * TROWS)])
                    ob_v[p, h, pl.ds(j * LANES, LANES)] = g

        def half(t, p, si, so):
            r = row0 + 2 * t + p
            pltpu.make_async_copy(x_hbm.at[0, r], x_v.at[p], si).wait()

            @pl.when(t > 0)
            def _():
                pltpu.make_async_copy(
                    ob_v.at[p], out_hbm.at[0, r - 2], so).wait()

            compute(p)
            pltpu.async_copy(ob_v.at[p], out_hbm.at[0, r], so)
            nxt = lax.min(r + 2, last)
            pltpu.async_copy(x_hbm.at[0, nxt], x_v.at[p], si)

        def body(t, carry):
            half(t, 0, si0, so0)
            half(t, 1, si1, so1)
            return carry

        lax.fori_loop(0, rows_per_w // 2, body, 0)

        pltpu.make_async_copy(x_hbm.at[0, last], x_v.at[0], si0).wait()
        pltpu.make_async_copy(x_hbm.at[0, last], x_v.at[1], si1).wait()
        pltpu.make_async_copy(ob_v.at[0], out_hbm.at[0, last - 1], so0).wait()
        pltpu.make_async_copy(ob_v.at[1], out_hbm.at[0, last], so1).wait()

    return k(d_mat, table_t_flat)


def kernel(d_mat, embeddings_table):
    table_t_flat = embeddings_table.T.reshape(-1)
    out_t = _embed(d_mat, table_t_flat)
    return jnp.transpose(out_t, (0, 1, 3, 2))


# speedup vs baseline: 1.1442x; 1.1442x over previous
import functools

import jax
import jax.numpy as jnp
from jax import lax
from jax.experimental import pallas as pl
from jax.experimental.pallas import tpu as pltpu
from jax.experimental.pallas import tpu_sc as plsc

N_POS_EMB = 512
N_HEADS = 16
TROWS = N_POS_EMB + 1

NC = 2
NS = 16
NW = NC * NS
LANES = 16


@jax.jit
def _embed(d_mat, table_t_flat):
    _, R, C = d_mat.shape
    rows_per_w = R // NW
    mesh = plsc.VectorSubcoreMesh(core_axis_name="c", subcore_axis_name="s")

    @functools.partial(
        pl.kernel,
        out_type=jax.ShapeDtypeStruct((1, R, N_HEADS, C), jnp.float32),
        mesh=mesh,
        scratch_types=[
            pltpu.VMEM((N_HEADS * TROWS,), jnp.float32),
            pltpu.VMEM((2, C), jnp.float32),
            pltpu.VMEM((2, N_HEADS, C), jnp.float32),
            pltpu.SemaphoreType.DMA,
            pltpu.SemaphoreType.DMA,
            pltpu.SemaphoreType.DMA,
            pltpu.SemaphoreType.DMA,
        ],
        compiler_params=pltpu.CompilerParams(needs_layout_passes=False),
    )
    def k(x_hbm, tab_hbm, out_hbm, tab_v, x_v, ob_v, si0, si1, so0, so1):
        wid = lax.axis_index("s") * NC + lax.axis_index("c")
        row0 = wid * rows_per_w
        last = row0 + rows_per_w - 1

        pltpu.sync_copy(tab_hbm, tab_v)
        pltpu.async_copy(x_hbm.at[0, row0], x_v.at[0], si0)
        pltpu.async_copy(x_hbm.at[0, row0 + 1], x_v.at[1], si1)

        def compute(p):
            @plsc.parallel_loop(0, C // LANES, 1, unroll=4)
            def _(j):
                x = x_v[p, pl.ds(j * LANES, LANES)]
                v = jnp.minimum(x, 1.0) * float(N_POS_EMB)
                idx = v.astype(jnp.int32)
                for h in range(N_HEADS):
                    g = plsc.load_gather(tab_v, [idx + (h * TROWS)])
                    ob_v[p, h, pl.ds(j * LANES, LANES)] = g

        def half(t, p, si, so):
            r = row0 + 2 * t + p
            pltpu.make_async_copy(x_hbm.at[0, r], x_v.at[p], si).wait()

            @pl.when(t > 0)
            def _():
                pltpu.make_async_copy(
                    ob_v.at[p], out_hbm.at[0, r - 2], so).wait()

            compute(p)
            pltpu.async_copy(ob_v.at[p], out_hbm.at[0, r], so)
            nxt = lax.min(r + 2, last)
            pltpu.async_copy(x_hbm.at[0, nxt], x_v.at[p], si)

        def body(t, carry):
            half(t, 0, si0, so0)
            half(t, 1, si1, so1)
            return carry

        lax.fori_loop(0, rows_per_w // 2, body, 0)

        pltpu.make_async_copy(x_hbm.at[0, last], x_v.at[0], si0).wait()
        pltpu.make_async_copy(x_hbm.at[0, last], x_v.at[1], si1).wait()
        pltpu.make_async_copy(ob_v.at[0], out_hbm.at[0, last - 1], so0).wait()
        pltpu.make_async_copy(ob_v.at[1], out_hbm.at[0, last], so1).wait()

    return k(d_mat, table_t_flat)


def kernel(d_mat, embeddings_table):
    table_t_flat = embeddings_table.T.reshape(-1)
    out_t = _embed(d_mat, table_t_flat)
    return jnp.transpose(out_t, (0, 1, 3, 2))
